# Initial kernel scaffold; baseline (speedup 1.0000x reference)
import jax
import jax.numpy as jnp
from jax.experimental import pallas as pl


def kernel(firings):
    raise NotImplementedError("write your pallas kernel here")


# speedup vs baseline: 3.5841x; 3.5841x over previous
import numpy as np
import jax
import jax.numpy as jnp
from jax.experimental import pallas as pl

_SEL = np.zeros((256, 128), dtype=np.float32)
_SEL[np.arange(0, 256, 2), np.arange(128)] = 1.0


def _select_even(x_ref, s_ref, o_ref):
    o_ref[...] = jax.lax.dot(
        x_ref[...], s_ref[...], precision=jax.lax.Precision.HIGHEST,
        preferred_element_type=jnp.float32)


def kernel(firings):
    B, R, C = firings.shape
    rows = B * R
    x = firings.reshape(rows, C)
    sel = jnp.asarray(_SEL)
    blk = 2048
    out = pl.pallas_call(
        _select_even,
        grid=(rows // blk,),
        in_specs=[
            pl.BlockSpec((blk, C), lambda i: (i, 0)),
            pl.BlockSpec((C, C // 2), lambda i: (0, 0)),
        ],
        out_specs=pl.BlockSpec((blk, C // 2), lambda i: (i, 0)),
        out_shape=jax.ShapeDtypeStruct((rows, C // 2), firings.dtype),
    )(x, sel)
    return out.reshape(B, R, C // 2)
